# baseline (speedup 1.0000x reference)
import functools

import jax
import jax.numpy as jnp
from jax import lax
from jax.experimental import pallas as pl
from jax.experimental.pallas import tpu as pltpu
from jax.experimental.pallas import tpu_sc as plsc


def _make_sc_gather(V2, D2, B):
    info = plsc.get_sparse_core_info()
    NC, NS = info.num_cores, info.num_subcores
    NW = NC * NS
    assert B % NW == 0
    b_per_w = B // NW
    CH = 512
    assert b_per_w % CH == 0
    n_chunks = b_per_w // CH
    mesh = plsc.VectorSubcoreMesh(core_axis_name="c", subcore_axis_name="s")

    @functools.partial(
        pl.kernel,
        out_type=jax.ShapeDtypeStruct((B, D2), jnp.float32),
        mesh=mesh,
        scratch_types=[
            pltpu.VMEM((CH,), jnp.int32),
            pltpu.VMEM((CH, D2), jnp.float32),
            pltpu.SemaphoreType.DMA,
        ],
    )
    def sc_gather(table_hbm, idx_hbm, out_hbm, idx_v, rows_v, sem):
        wid = lax.axis_index("s") * NC + lax.axis_index("c")
        base = wid * b_per_w

        def body(c, carry):
            off = base + c * CH
            pltpu.sync_copy(idx_hbm.at[pl.ds(off, CH)], idx_v)
            pltpu.async_copy(table_hbm.at[idx_v], rows_v, sem).wait()
            pltpu.sync_copy(rows_v, out_hbm.at[pl.ds(off, CH)])
            return carry

        lax.fori_loop(0, n_chunks, body, 0)

    return sc_gather


def _make_tc_project(Bt, L, D2, D_out):
    BB = 256
    assert Bt % BB == 0
    BM = BB * L

    def mm_body(x_ref, par_ref, wlo_ref, whi_ref, b_ref, o_ref):
        x = x_ref[...]
        y_lo = jnp.dot(x, wlo_ref[...], preferred_element_type=jnp.float32,
                       precision=lax.Precision.HIGHEST)
        y_hi = jnp.dot(x, whi_ref[...], preferred_element_type=jnp.float32,
                       precision=lax.Precision.HIGHEST)
        ylo3 = y_lo.reshape(BB, L, D_out)
        yhi3 = y_hi.reshape(BB, L, D_out)
        par = par_ref[...]
        o_ref[...] = ylo3 + par[:, :, None] * (yhi3 - ylo3) + b_ref[...][None]

    return pl.pallas_call(
        mm_body,
        grid=(Bt // BB,),
        in_specs=[
            pl.BlockSpec((BM, D2), lambda i: (i, 0)),
            pl.BlockSpec((BB, L), lambda i: (i, 0)),
            pl.BlockSpec((D2, D_out), lambda i: (0, 0)),
            pl.BlockSpec((D2, D_out), lambda i: (0, 0)),
            pl.BlockSpec((1, D_out), lambda i: (0, 0)),
        ],
        out_specs=pl.BlockSpec((BB, L, D_out), lambda i: (i, 0, 0)),
        out_shape=jax.ShapeDtypeStruct((Bt, L, D_out), jnp.float32),
        compiler_params=pltpu.CompilerParams(
            dimension_semantics=("arbitrary",),
        ),
    )


def kernel(indices, table, W, b):
    Bt, L = indices.shape
    V, D = table.shape
    D_out = W.shape[1]
    Bf = Bt * L
    idx = indices.astype(jnp.int32)
    idx2 = (idx >> 1).reshape(Bf)
    par = (idx & 1).astype(jnp.float32)
    tp = table.reshape(V // 2, 2 * D)
    zeros = jnp.zeros_like(W)
    w_lo = jnp.concatenate([W, zeros], axis=0)
    w_hi = jnp.concatenate([zeros, W], axis=0)
    g = _make_sc_gather(V // 2, 2 * D, Bf)(tp, idx2)
    out = _make_tc_project(Bt, L, 2 * D, D_out)(
        g, par, w_lo, w_hi, b.reshape(1, D_out))
    return out


# speedup vs baseline: 1.3904x; 1.3904x over previous
import functools

import jax
import jax.numpy as jnp
from jax import lax
from jax.experimental import pallas as pl
from jax.experimental.pallas import tpu as pltpu
from jax.experimental.pallas import tpu_sc as plsc


def _make_tc_transform(V, D, D_out):
    BK = 8192
    grid = (V + BK - 1) // BK

    def body(x_ref, w_ref, b_ref, o_ref):
        x = x_ref[...]
        yt = lax.dot_general(x, w_ref[...], (((0,), (0,)), ((), ())),
                             preferred_element_type=jnp.float32,
                             precision=lax.Precision.HIGHEST)
        yt = yt + b_ref[...]
        o_ref[...] = jnp.concatenate([yt, yt], axis=1)

    return pl.pallas_call(
        body,
        grid=(grid,),
        in_specs=[
            pl.BlockSpec((D, BK), lambda j: (0, j)),
            pl.BlockSpec((D, D_out), lambda j: (0, 0)),
            pl.BlockSpec((1, D_out), lambda j: (0, 0)),
        ],
        out_specs=pl.BlockSpec((BK, 2 * D_out), lambda j: (j, 0)),
        out_shape=jax.ShapeDtypeStruct((V, 2 * D_out), jnp.float32),
        compiler_params=pltpu.CompilerParams(
            dimension_semantics=("arbitrary",),
        ),
    )


def _make_sc_gather(V, D2, B):
    info = plsc.get_sparse_core_info()
    NC, NS = info.num_cores, info.num_subcores
    NW = NC * NS
    assert B % NW == 0
    b_per_w = B // NW
    CH = 512
    assert b_per_w % CH == 0
    n_chunks = b_per_w // CH
    mesh = plsc.VectorSubcoreMesh(core_axis_name="c", subcore_axis_name="s")

    @functools.partial(
        pl.kernel,
        out_type=jax.ShapeDtypeStruct((B, D2), jnp.float32),
        mesh=mesh,
        scratch_types=[
            pltpu.VMEM((CH,), jnp.int32),
            pltpu.VMEM((CH, D2), jnp.float32),
            pltpu.SemaphoreType.DMA,
        ],
    )
    def sc_gather(table_hbm, idx_hbm, out_hbm, idx_v, rows_v, sem):
        wid = lax.axis_index("s") * NC + lax.axis_index("c")
        base = wid * b_per_w

        def body(c, carry):
            off = base + c * CH
            pltpu.sync_copy(idx_hbm.at[pl.ds(off, CH)], idx_v)
            pltpu.async_copy(table_hbm.at[idx_v], rows_v, sem).wait()
            pltpu.sync_copy(rows_v, out_hbm.at[pl.ds(off, CH)])
            return carry

        lax.fori_loop(0, n_chunks, body, 0)

    return sc_gather


def _make_tc_emit(Bt, L, D_out):
    BB = 512
    assert Bt % BB == 0
    nj = Bt // BB

    def body(x_ref, o_ref):
        x = x_ref[...]
        xt = x.T
        o_ref[...] = xt[None, :D_out, :]

    return pl.pallas_call(
        body,
        grid=(L, nj),
        in_specs=[
            pl.BlockSpec((BB, 2 * D_out), lambda l, j: (l * nj + j, 0)),
        ],
        out_specs=pl.BlockSpec((1, D_out, BB), lambda l, j: (l, 0, j)),
        out_shape=jax.ShapeDtypeStruct((L, D_out, Bt), jnp.float32),
        compiler_params=pltpu.CompilerParams(
            dimension_semantics=("arbitrary", "arbitrary"),
        ),
    )


def kernel(indices, table, W, b):
    Bt, L = indices.shape
    V, D = table.shape
    D_out = W.shape[1]
    Bf = Bt * L
    tT = table.T
    twb = _make_tc_transform(V, D, D_out)(tT, W, b.reshape(1, D_out))
    idxn = indices.T.reshape(Bf).astype(jnp.int32)
    g = _make_sc_gather(V, 2 * D_out, Bf)(twb, idxn)
    out_t = _make_tc_emit(Bt, L, D_out)(g)
    return jnp.transpose(out_t, (2, 0, 1))


# speedup vs baseline: 2.2527x; 1.6201x over previous
import functools

import jax
import jax.numpy as jnp
from jax import lax
from jax.experimental import pallas as pl
from jax.experimental.pallas import tpu as pltpu
from jax.experimental.pallas import tpu_sc as plsc


def _make_tc_transform(V, D, D_out):
    BK = 8192
    grid = (V + BK - 1) // BK

    def body(x_ref, w_ref, b_ref, o_ref):
        x = x_ref[...]
        yt = lax.dot_general(x, w_ref[...], (((0,), (0,)), ((), ())),
                             preferred_element_type=jnp.float32)
        yt = yt + b_ref[...]
        o_ref[...] = jnp.concatenate([yt, yt], axis=1)

    return pl.pallas_call(
        body,
        grid=(grid,),
        in_specs=[
            pl.BlockSpec((D, BK), lambda j: (0, j)),
            pl.BlockSpec((D, D_out), lambda j: (0, 0)),
            pl.BlockSpec((1, D_out), lambda j: (0, 0)),
        ],
        out_specs=pl.BlockSpec((BK, 2 * D_out), lambda j: (j, 0)),
        out_shape=jax.ShapeDtypeStruct((V, 2 * D_out), jnp.float32),
        compiler_params=pltpu.CompilerParams(
            dimension_semantics=("arbitrary",),
        ),
    )


def _make_sc_gather(V, D2, B):
    info = plsc.get_sparse_core_info()
    NC, NS = info.num_cores, info.num_subcores
    NW = NC * NS
    assert B % NW == 0
    b_per_w = B // NW
    CH = 512
    assert b_per_w % CH == 0
    n_chunks = b_per_w // CH
    mesh = plsc.VectorSubcoreMesh(core_axis_name="c", subcore_axis_name="s")

    @functools.partial(
        pl.kernel,
        out_type=jax.ShapeDtypeStruct((B, D2), jnp.float32),
        mesh=mesh,
        scratch_types=[
            pltpu.VMEM((CH,), jnp.int32),
            pltpu.VMEM((CH, D2), jnp.float32),
            pltpu.SemaphoreType.DMA,
        ],
    )
    def sc_gather(table_hbm, idx_hbm, out_hbm, idx_v, rows_v, sem):
        wid = lax.axis_index("s") * NC + lax.axis_index("c")
        base = wid * b_per_w

        def body(c, carry):
            off = base + c * CH
            pltpu.sync_copy(idx_hbm.at[pl.ds(off, CH)], idx_v)
            pltpu.async_copy(table_hbm.at[idx_v], rows_v, sem).wait()
            pltpu.sync_copy(rows_v, out_hbm.at[pl.ds(off, CH)])
            return carry

        lax.fori_loop(0, n_chunks, body, 0)

    return sc_gather


def _make_tc_emit(Bt, L, D_out):
    BB = 2048
    assert Bt % BB == 0
    nj = Bt // BB

    def body(x_ref, o_ref):
        x = x_ref[...]
        xt = x.T
        o_ref[...] = xt[None, :D_out, :]

    return pl.pallas_call(
        body,
        grid=(L, nj),
        in_specs=[
            pl.BlockSpec((BB, 2 * D_out), lambda l, j: (l * nj + j, 0)),
        ],
        out_specs=pl.BlockSpec((1, D_out, BB), lambda l, j: (l, 0, j)),
        out_shape=jax.ShapeDtypeStruct((L, D_out, Bt), jnp.float32),
        compiler_params=pltpu.CompilerParams(
            dimension_semantics=("arbitrary", "arbitrary"),
        ),
    )


def kernel(indices, table, W, b):
    Bt, L = indices.shape
    V, D = table.shape
    D_out = W.shape[1]
    Bf = Bt * L
    tT = table.T
    twb = _make_tc_transform(V, D, D_out)(tT, W, b.reshape(1, D_out))
    idxn = indices.T.reshape(Bf).astype(jnp.int32)
    g = _make_sc_gather(V, 2 * D_out, Bf)(twb, idxn)
    out_t = _make_tc_emit(Bt, L, D_out)(g)
    return jnp.transpose(out_t, (2, 0, 1))


# speedup vs baseline: 2.5000x; 1.1098x over previous
import functools

import jax
import jax.numpy as jnp
from jax import lax
from jax.experimental import pallas as pl
from jax.experimental.pallas import tpu as pltpu
from jax.experimental.pallas import tpu_sc as plsc


def _make_tc_transform(V, D, D_out):
    BK = 8192
    grid = (V + BK - 1) // BK

    def body(x_ref, w_ref, b_ref, o_ref):
        x = x_ref[...]
        yt = lax.dot_general(x, w_ref[...], (((0,), (0,)), ((), ())),
                             preferred_element_type=jnp.float32)
        yt = yt + b_ref[...]
        o_ref[...] = jnp.concatenate([yt, yt], axis=1)

    return pl.pallas_call(
        body,
        grid=(grid,),
        in_specs=[
            pl.BlockSpec((D, BK), lambda j: (0, j)),
            pl.BlockSpec((D, D_out), lambda j: (0, 0)),
            pl.BlockSpec((1, D_out), lambda j: (0, 0)),
        ],
        out_specs=pl.BlockSpec((BK, 2 * D_out), lambda j: (j, 0)),
        out_shape=jax.ShapeDtypeStruct((V, 2 * D_out), jnp.float32),
        compiler_params=pltpu.CompilerParams(
            dimension_semantics=("arbitrary",),
        ),
    )


def _make_sc_gather(V, D2, B):
    info = plsc.get_sparse_core_info()
    NC, NS = info.num_cores, info.num_subcores
    NW = NC * NS
    assert B % NW == 0
    b_per_w = B // NW
    CH = 416
    assert b_per_w % (2 * CH) == 0
    n_pairs = b_per_w // (2 * CH)
    mesh = plsc.VectorSubcoreMesh(core_axis_name="c", subcore_axis_name="s")

    @functools.partial(
        pl.kernel,
        out_type=jax.ShapeDtypeStruct((B, D2), jnp.float32),
        mesh=mesh,
        scratch_types=[
            pltpu.VMEM((CH,), jnp.int32),
            pltpu.VMEM((CH,), jnp.int32),
            pltpu.VMEM((CH, D2), jnp.float32),
            pltpu.VMEM((CH, D2), jnp.float32),
            pltpu.SemaphoreType.DMA,
            pltpu.SemaphoreType.DMA,
        ],
    )
    def sc_gather(table_hbm, idx_hbm, out_hbm, i0, i1, r0, r1, s0, s1):
        wid = lax.axis_index("s") * NC + lax.axis_index("c")
        base = wid * b_per_w

        pltpu.sync_copy(idx_hbm.at[pl.ds(base, CH)], i0)
        g0 = pltpu.async_copy(table_hbm.at[i0], r0, s0)
        pltpu.sync_copy(idx_hbm.at[pl.ds(base + CH, CH)], i1)
        g1 = pltpu.async_copy(table_hbm.at[i1], r1, s1)

        def body(p, carry):
            ca = base + 2 * p * CH
            g0.wait()
            pltpu.sync_copy(r0, out_hbm.at[pl.ds(ca, CH)])

            @pl.when(p + 1 < n_pairs)
            def _():
                pltpu.sync_copy(idx_hbm.at[pl.ds(ca + 2 * CH, CH)], i0)
                pltpu.async_copy(table_hbm.at[i0], r0, s0)

            g1.wait()
            pltpu.sync_copy(r1, out_hbm.at[pl.ds(ca + CH, CH)])

            @pl.when(p + 1 < n_pairs)
            def _():
                pltpu.sync_copy(idx_hbm.at[pl.ds(ca + 3 * CH, CH)], i1)
                pltpu.async_copy(table_hbm.at[i1], r1, s1)

            return carry

        lax.fori_loop(0, n_pairs, body, 0)

    return sc_gather


def _make_tc_emit(Bt, L, D_out):
    BB = 4096
    assert Bt % BB == 0
    nj = Bt // BB

    def body(x_ref, o_ref):
        x = x_ref[...]
        o_ref[...] = x.T[None, :D_out, :]

    return pl.pallas_call(
        body,
        grid=(L, nj),
        in_specs=[
            pl.BlockSpec((BB, 2 * D_out), lambda l, j: (l * nj + j, 0)),
        ],
        out_specs=pl.BlockSpec((1, D_out, BB), lambda l, j: (l, 0, j)),
        out_shape=jax.ShapeDtypeStruct((L, D_out, Bt), jnp.float32),
        compiler_params=pltpu.CompilerParams(
            dimension_semantics=("arbitrary", "arbitrary"),
        ),
    )


def kernel(indices, table, W, b):
    Bt, L = indices.shape
    V, D = table.shape
    D_out = W.shape[1]
    Bf = Bt * L
    tT = table.T
    twb = _make_tc_transform(V, D, D_out)(tT, W, b.reshape(1, D_out))
    idxn = indices.T.reshape(Bf).astype(jnp.int32)
    g = _make_sc_gather(V, 2 * D_out, Bf)(twb, idxn)
    out_t = _make_tc_emit(Bt, L, D_out)(g)
    return jnp.transpose(out_t, (2, 0, 1))


# speedup vs baseline: 2.8333x; 1.1333x over previous
import functools

import jax
import jax.numpy as jnp
from jax import lax
from jax.experimental import pallas as pl
from jax.experimental.pallas import tpu as pltpu
from jax.experimental.pallas import tpu_sc as plsc


def _make_tc_transform(V, D, D_out):
    BK = 16384
    grid = (V + BK - 1) // BK

    def body(x_ref, w2_ref, b2_ref, o_ref):
        x = x_ref[...]
        yt = lax.dot_general(x, w2_ref[...], (((0,), (0,)), ((), ())),
                             preferred_element_type=jnp.float32)
        o_ref[...] = yt + b2_ref[...]

    return pl.pallas_call(
        body,
        grid=(grid,),
        in_specs=[
            pl.BlockSpec((D, BK), lambda j: (0, j)),
            pl.BlockSpec((D, 2 * D_out), lambda j: (0, 0)),
            pl.BlockSpec((1, 2 * D_out), lambda j: (0, 0)),
        ],
        out_specs=pl.BlockSpec((BK, 2 * D_out), lambda j: (j, 0)),
        out_shape=jax.ShapeDtypeStruct((V, 2 * D_out), jnp.float32),
        compiler_params=pltpu.CompilerParams(
            dimension_semantics=("arbitrary",),
        ),
    )


def _make_sc_gather(V, D2, B):
    info = plsc.get_sparse_core_info()
    NC, NS = info.num_cores, info.num_subcores
    NW = NC * NS
    assert B % NW == 0
    b_per_w = B // NW
    CH = 416
    assert b_per_w % (2 * CH) == 0
    n_pairs = b_per_w // (2 * CH)
    mesh = plsc.VectorSubcoreMesh(core_axis_name="c", subcore_axis_name="s")

    @functools.partial(
        pl.kernel,
        out_type=jax.ShapeDtypeStruct((B, D2), jnp.float32),
        mesh=mesh,
        scratch_types=[
            pltpu.VMEM((CH,), jnp.int32),
            pltpu.VMEM((CH,), jnp.int32),
            pltpu.VMEM((CH, D2), jnp.float32),
            pltpu.VMEM((CH, D2), jnp.float32),
            pltpu.SemaphoreType.DMA,
            pltpu.SemaphoreType.DMA,
        ],
    )
    def sc_gather(table_hbm, idx_hbm, out_hbm, i0, i1, r0, r1, s0, s1):
        wid = lax.axis_index("s") * NC + lax.axis_index("c")
        base = wid * b_per_w

        pltpu.sync_copy(idx_hbm.at[pl.ds(base, CH)], i0)
        g0 = pltpu.async_copy(table_hbm.at[i0], r0, s0)
        pltpu.sync_copy(idx_hbm.at[pl.ds(base + CH, CH)], i1)
        g1 = pltpu.async_copy(table_hbm.at[i1], r1, s1)

        def body(p, carry):
            ca = base + 2 * p * CH
            g0.wait()
            pltpu.sync_copy(r0, out_hbm.at[pl.ds(ca, CH)])

            @pl.when(p + 1 < n_pairs)
            def _():
                pltpu.sync_copy(idx_hbm.at[pl.ds(ca + 2 * CH, CH)], i0)
                pltpu.async_copy(table_hbm.at[i0], r0, s0)

            g1.wait()
            pltpu.sync_copy(r1, out_hbm.at[pl.ds(ca + CH, CH)])

            @pl.when(p + 1 < n_pairs)
            def _():
                pltpu.sync_copy(idx_hbm.at[pl.ds(ca + 3 * CH, CH)], i1)
                pltpu.async_copy(table_hbm.at[i1], r1, s1)

            return carry

        lax.fori_loop(0, n_pairs, body, 0)

    return sc_gather


def _make_tc_emit(Bt, L, D_out):
    BB = 4096
    assert Bt % BB == 0
    nj = Bt // BB

    def body(x_ref, o_ref):
        x = x_ref[...]
        o_ref[...] = x.T[None, :D_out, :]

    return pl.pallas_call(
        body,
        grid=(L, nj),
        in_specs=[
            pl.BlockSpec((BB, 2 * D_out), lambda l, j: (l * nj + j, 0)),
        ],
        out_specs=pl.BlockSpec((1, D_out, BB), lambda l, j: (l, 0, j)),
        out_shape=jax.ShapeDtypeStruct((L, D_out, Bt), jnp.float32),
        compiler_params=pltpu.CompilerParams(
            dimension_semantics=("arbitrary", "arbitrary"),
        ),
    )


def kernel(indices, table, W, b):
    Bt, L = indices.shape
    V, D = table.shape
    D_out = W.shape[1]
    Bf = Bt * L
    tT = table.T
    W2 = jnp.concatenate([W, W], axis=1)
    b2 = jnp.concatenate([b, b]).reshape(1, 2 * D_out)
    twb = _make_tc_transform(V, D, D_out)(tT, W2, b2)
    idxn = indices.T.reshape(Bf).astype(jnp.int32)
    g = _make_sc_gather(V, 2 * D_out, Bf)(twb, idxn)
    out_t = _make_tc_emit(Bt, L, D_out)(g)
    return jnp.transpose(out_t, (2, 0, 1))


# speedup vs baseline: 2.9217x; 1.0312x over previous
import functools

import jax
import jax.numpy as jnp
from jax import lax
from jax.experimental import pallas as pl
from jax.experimental.pallas import tpu as pltpu
from jax.experimental.pallas import tpu_sc as plsc


def _make_tc_transform(V, D, D_out):
    BK = 16384
    grid = (V + BK - 1) // BK

    def body(x_ref, w2_ref, b2_ref, o_ref):
        x = x_ref[...]
        yt = lax.dot_general(x, w2_ref[...], (((0,), (0,)), ((), ())),
                             preferred_element_type=jnp.float32)
        o_ref[...] = yt + b2_ref[...]

    return pl.pallas_call(
        body,
        grid=(grid,),
        in_specs=[
            pl.BlockSpec((D, BK), lambda j: (0, j)),
            pl.BlockSpec((D, 2 * D_out), lambda j: (0, 0)),
            pl.BlockSpec((1, 2 * D_out), lambda j: (0, 0)),
        ],
        out_specs=pl.BlockSpec((BK, 2 * D_out), lambda j: (j, 0)),
        out_shape=jax.ShapeDtypeStruct((V, 2 * D_out), jnp.float32),
        compiler_params=pltpu.CompilerParams(
            dimension_semantics=("arbitrary",),
        ),
    )


def _make_sc_gather(V, D2, B):
    info = plsc.get_sparse_core_info()
    NC, NS = info.num_cores, info.num_subcores
    NW = NC * NS
    assert B % NW == 0
    b_per_w = B // NW
    CH = 416
    assert b_per_w % (2 * CH) == 0
    n_pairs = b_per_w // (2 * CH)
    mesh = plsc.VectorSubcoreMesh(core_axis_name="c", subcore_axis_name="s")

    @functools.partial(
        pl.kernel,
        out_type=jax.ShapeDtypeStruct((B, D2), jnp.float32),
        mesh=mesh,
        scratch_types=[
            pltpu.VMEM((CH,), jnp.int32),
            pltpu.VMEM((CH,), jnp.int32),
            pltpu.VMEM((CH, D2), jnp.float32),
            pltpu.VMEM((CH, D2), jnp.float32),
            pltpu.SemaphoreType.DMA,
            pltpu.SemaphoreType.DMA,
        ],
    )
    def sc_gather(table_hbm, idx_hbm, out_hbm, i0, i1, r0, r1, s0, s1):
        wid = lax.axis_index("s") * NC + lax.axis_index("c")
        base = wid * b_per_w

        pltpu.sync_copy(idx_hbm.at[pl.ds(base, CH)], i0)
        g0 = pltpu.async_copy(table_hbm.at[i0], r0, s0)
        pltpu.sync_copy(idx_hbm.at[pl.ds(base + CH, CH)], i1)
        g1 = pltpu.async_copy(table_hbm.at[i1], r1, s1)

        def body(p, carry):
            ca = base + 2 * p * CH
            g0.wait()
            pltpu.sync_copy(r0, out_hbm.at[pl.ds(ca, CH)])

            @pl.when(p + 1 < n_pairs)
            def _():
                pltpu.sync_copy(idx_hbm.at[pl.ds(ca + 2 * CH, CH)], i0)
                pltpu.async_copy(table_hbm.at[i0], r0, s0)

            g1.wait()
            pltpu.sync_copy(r1, out_hbm.at[pl.ds(ca + CH, CH)])

            @pl.when(p + 1 < n_pairs)
            def _():
                pltpu.sync_copy(idx_hbm.at[pl.ds(ca + 3 * CH, CH)], i1)
                pltpu.async_copy(table_hbm.at[i1], r1, s1)

            return carry

        lax.fori_loop(0, n_pairs, body, 0)

    return sc_gather


def _make_tc_emit_chunk(Bt, L, Lh, l0, D_out, aliased):
    BB = 4096
    assert Bt % BB == 0
    nj = Bt // BB

    if aliased:
        def body(prev_ref, x_ref, o_ref):
            x = x_ref[...]
            o_ref[...] = x.T[None, :D_out, :]

        in_specs = [
            pl.BlockSpec((1, 8, 128), lambda l, j: (0, 0, 0)),
            pl.BlockSpec((BB, 2 * D_out), lambda l, j: (l * nj + j, 0)),
        ]
        io_alias = {0: 0}
    else:
        def body(x_ref, o_ref):
            x = x_ref[...]
            o_ref[...] = x.T[None, :D_out, :]

        in_specs = [
            pl.BlockSpec((BB, 2 * D_out), lambda l, j: (l * nj + j, 0)),
        ]
        io_alias = {}

    return pl.pallas_call(
        body,
        grid=(Lh, nj),
        in_specs=in_specs,
        out_specs=pl.BlockSpec((1, D_out, BB), lambda l, j: (l + l0, 0, j)),
        out_shape=jax.ShapeDtypeStruct((L, D_out, Bt), jnp.float32),
        input_output_aliases=io_alias,
        compiler_params=pltpu.CompilerParams(
            dimension_semantics=("arbitrary", "arbitrary"),
        ),
    )


def kernel(indices, table, W, b):
    Bt, L = indices.shape
    V, D = table.shape
    D_out = W.shape[1]
    Bf = Bt * L
    NCH = 2
    Lh = L // NCH
    Bh = Bt * Lh
    tT = table.T
    W2 = jnp.concatenate([W, W], axis=1)
    b2 = jnp.concatenate([b, b]).reshape(1, 2 * D_out)
    twb = _make_tc_transform(V, D, D_out)(tT, W2, b2)
    idxn = indices.T.reshape(Bf).astype(jnp.int32)
    gather = _make_sc_gather(V, 2 * D_out, Bh)
    gs = [gather(twb, lax.slice(idxn, (k * Bh,), ((k + 1) * Bh,)))
          for k in range(NCH)]
    out = _make_tc_emit_chunk(Bt, L, Lh, 0, D_out, aliased=False)(gs[0])
    for k in range(1, NCH):
        out = _make_tc_emit_chunk(Bt, L, Lh, k * Lh, D_out, aliased=True)(
            out, gs[k])
    return jnp.transpose(out, (2, 0, 1))


# speedup vs baseline: 3.4026x; 1.1646x over previous
import functools

import jax
import jax.numpy as jnp
from jax import lax
from jax.experimental import pallas as pl
from jax.experimental.pallas import tpu as pltpu
from jax.experimental.pallas import tpu_sc as plsc

_BK = 8192


def _make_tc_transform(V, D, D_out, split):
    nlo = split // _BK
    R = V - split
    grid = (R + _BK - 1) // _BK

    def body(xlo_ref, xhi_ref, w4_ref, b2_ref, o_ref):
        xcat = jnp.concatenate([xlo_ref[...], xhi_ref[...]], axis=0)
        yt = lax.dot_general(xcat, w4_ref[...], (((0,), (0,)), ((), ())),
                             preferred_element_type=jnp.float32)
        o_ref[...] = yt + b2_ref[...]

    return pl.pallas_call(
        body,
        grid=(grid,),
        in_specs=[
            pl.BlockSpec((D, _BK), lambda j: (0, j)),
            pl.BlockSpec((D, _BK), lambda j: (0, j + nlo)),
            pl.BlockSpec((2 * D, 2 * D_out), lambda j: (0, 0)),
            pl.BlockSpec((1, 2 * D_out), lambda j: (0, 0)),
        ],
        out_specs=pl.BlockSpec((_BK, 2 * D_out), lambda j: (j, 0)),
        out_shape=jax.ShapeDtypeStruct((R, 2 * D_out), jnp.float32),
        compiler_params=pltpu.CompilerParams(
            dimension_semantics=("arbitrary",),
        ),
    )


def _make_sc_gather(D2, B):
    info = plsc.get_sparse_core_info()
    NC, NS = info.num_cores, info.num_subcores
    NW = NC * NS
    assert B % NW == 0
    b_per_w = B // NW
    CH = 416
    assert b_per_w % (2 * CH) == 0
    n_pairs = b_per_w // (2 * CH)
    mesh = plsc.VectorSubcoreMesh(core_axis_name="c", subcore_axis_name="s")

    @functools.partial(
        pl.kernel,
        out_type=jax.ShapeDtypeStruct((B, D2), jnp.float32),
        mesh=mesh,
        scratch_types=[
            pltpu.VMEM((CH,), jnp.int32),
            pltpu.VMEM((CH,), jnp.int32),
            pltpu.VMEM((CH, D2), jnp.float32),
            pltpu.VMEM((CH, D2), jnp.float32),
            pltpu.SemaphoreType.DMA,
            pltpu.SemaphoreType.DMA,
        ],
    )
    def sc_gather(table_hbm, idx_hbm, out_hbm, i0, i1, r0, r1, s0, s1):
        wid = lax.axis_index("s") * NC + lax.axis_index("c")
        base = wid * b_per_w

        pltpu.sync_copy(idx_hbm.at[pl.ds(base, CH)], i0)
        g0 = pltpu.async_copy(table_hbm.at[i0], r0, s0)
        pltpu.sync_copy(idx_hbm.at[pl.ds(base + CH, CH)], i1)
        g1 = pltpu.async_copy(table_hbm.at[i1], r1, s1)

        def body(p, carry):
            ca = base + 2 * p * CH
            g0.wait()
            pltpu.sync_copy(r0, out_hbm.at[pl.ds(ca, CH)])

            @pl.when(p + 1 < n_pairs)
            def _():
                pltpu.sync_copy(idx_hbm.at[pl.ds(ca + 2 * CH, CH)], i0)
                pltpu.async_copy(table_hbm.at[i0], r0, s0)

            g1.wait()
            pltpu.sync_copy(r1, out_hbm.at[pl.ds(ca + CH, CH)])

            @pl.when(p + 1 < n_pairs)
            def _():
                pltpu.sync_copy(idx_hbm.at[pl.ds(ca + 3 * CH, CH)], i1)
                pltpu.async_copy(table_hbm.at[i1], r1, s1)

            return carry

        lax.fori_loop(0, n_pairs, body, 0)

    return sc_gather


def _make_tc_emit_chunk(Bt, L, Lh, l0, D_out, aliased):
    BB = 4096
    assert Bt % BB == 0
    nj = Bt // BB

    def select_t(x, h):
        xt = x.T
        lo = xt[:D_out, :]
        hi = xt[D_out:, :]
        return (lo + h * (hi - lo))[None]

    if aliased:
        def body(prev_ref, x_ref, h_ref, o_ref):
            o_ref[...] = select_t(x_ref[...], h_ref[0])

        in_specs = [
            pl.BlockSpec((1, 8, 128), lambda l, j: (0, 0, 0)),
            pl.BlockSpec((BB, 2 * D_out), lambda l, j: (l * nj + j, 0)),
            pl.BlockSpec((1, 1, BB), lambda l, j: (l + l0, 0, j)),
        ]
        io_alias = {0: 0}
    else:
        def body(x_ref, h_ref, o_ref):
            o_ref[...] = select_t(x_ref[...], h_ref[0])

        in_specs = [
            pl.BlockSpec((BB, 2 * D_out), lambda l, j: (l * nj + j, 0)),
            pl.BlockSpec((1, 1, BB), lambda l, j: (l + l0, 0, j)),
        ]
        io_alias = {}

    return pl.pallas_call(
        body,
        grid=(Lh, nj),
        in_specs=in_specs,
        out_specs=pl.BlockSpec((1, D_out, BB), lambda l, j: (l + l0, 0, j)),
        out_shape=jax.ShapeDtypeStruct((L, D_out, Bt), jnp.float32),
        input_output_aliases=io_alias,
        compiler_params=pltpu.CompilerParams(
            dimension_semantics=("arbitrary", "arbitrary"),
        ),
    )


def kernel(indices, table, W, b):
    Bt, L = indices.shape
    V, D = table.shape
    D_out = W.shape[1]
    Bf = Bt * L
    NCH = 2
    Lh = L // NCH
    Bh = Bt * Lh
    split = (V // 2 // _BK) * _BK
    tT = table.T
    zero = jnp.zeros_like(W)
    W4 = jnp.concatenate(
        [jnp.concatenate([W, zero], axis=1),
         jnp.concatenate([zero, W], axis=1)], axis=0)
    b2 = jnp.concatenate([b, b]).reshape(1, 2 * D_out)
    twb = _make_tc_transform(V, D, D_out, split)(tT, tT, W4, b2)
    idxT = indices.T.astype(jnp.int32)
    idx2 = jnp.where(idxT < split, idxT, idxT - split).reshape(Bf)
    half = (idxT >= split).astype(jnp.float32).reshape(L, 1, Bt)
    gather = _make_sc_gather(2 * D_out, Bh)
    gs = [gather(twb, lax.slice(idx2, (k * Bh,), ((k + 1) * Bh,)))
          for k in range(NCH)]
    out = _make_tc_emit_chunk(Bt, L, Lh, 0, D_out, aliased=False)(
        gs[0], half)
    for k in range(1, NCH):
        out = _make_tc_emit_chunk(Bt, L, Lh, k * Lh, D_out, aliased=True)(
            out, gs[k], half)
    return jnp.transpose(out, (2, 0, 1))


# speedup vs baseline: 3.4348x; 1.0095x over previous
import functools

import jax
import jax.numpy as jnp
from jax import lax
from jax.experimental import pallas as pl
from jax.experimental.pallas import tpu as pltpu
from jax.experimental.pallas import tpu_sc as plsc

_BK = 8192


def _make_tc_transform(V, D, D_out, split):
    nlo = split // _BK
    R = V - split
    grid = (R + _BK - 1) // _BK

    def body(xlo_ref, xhi_ref, w4_ref, b2_ref, o_ref):
        xcat = jnp.concatenate([xlo_ref[...], xhi_ref[...]], axis=0)
        yt = lax.dot_general(xcat, w4_ref[...], (((0,), (0,)), ((), ())),
                             preferred_element_type=jnp.float32)
        o_ref[...] = yt + b2_ref[...]

    return pl.pallas_call(
        body,
        grid=(grid,),
        in_specs=[
            pl.BlockSpec((D, _BK), lambda j: (0, j)),
            pl.BlockSpec((D, _BK), lambda j: (0, j + nlo)),
            pl.BlockSpec((2 * D, 2 * D_out), lambda j: (0, 0)),
            pl.BlockSpec((1, 2 * D_out), lambda j: (0, 0)),
        ],
        out_specs=pl.BlockSpec((_BK, 2 * D_out), lambda j: (j, 0)),
        out_shape=jax.ShapeDtypeStruct((R, 2 * D_out), jnp.float32),
        compiler_params=pltpu.CompilerParams(
            dimension_semantics=("arbitrary",),
        ),
    )


def _make_sc_gather(D2, B):
    info = plsc.get_sparse_core_info()
    NC, NS = info.num_cores, info.num_subcores
    NW = NC * NS
    assert B % NW == 0
    b_per_w = B // NW
    CH = 416
    assert b_per_w % (2 * CH) == 0
    n_pairs = b_per_w // (2 * CH)
    mesh = plsc.VectorSubcoreMesh(core_axis_name="c", subcore_axis_name="s")

    @functools.partial(
        pl.kernel,
        out_type=jax.ShapeDtypeStruct((B, D2), jnp.float32),
        mesh=mesh,
        scratch_types=[
            pltpu.VMEM((CH,), jnp.int32),
            pltpu.VMEM((CH,), jnp.int32),
            pltpu.VMEM((CH, D2), jnp.float32),
            pltpu.VMEM((CH, D2), jnp.float32),
            pltpu.SemaphoreType.DMA,
            pltpu.SemaphoreType.DMA,
        ],
    )
    def sc_gather(table_hbm, idx_hbm, out_hbm, i0, i1, r0, r1, s0, s1):
        wid = lax.axis_index("s") * NC + lax.axis_index("c")
        base = wid * b_per_w

        pltpu.sync_copy(idx_hbm.at[pl.ds(base, CH)], i0)
        g0 = pltpu.async_copy(table_hbm.at[i0], r0, s0)
        pltpu.sync_copy(idx_hbm.at[pl.ds(base + CH, CH)], i1)
        g1 = pltpu.async_copy(table_hbm.at[i1], r1, s1)

        def body(p, carry):
            ca = base + 2 * p * CH
            g0.wait()
            pltpu.sync_copy(r0, out_hbm.at[pl.ds(ca, CH)])

            @pl.when(p + 1 < n_pairs)
            def _():
                pltpu.sync_copy(idx_hbm.at[pl.ds(ca + 2 * CH, CH)], i0)
                pltpu.async_copy(table_hbm.at[i0], r0, s0)

            g1.wait()
            pltpu.sync_copy(r1, out_hbm.at[pl.ds(ca + CH, CH)])

            @pl.when(p + 1 < n_pairs)
            def _():
                pltpu.sync_copy(idx_hbm.at[pl.ds(ca + 3 * CH, CH)], i1)
                pltpu.async_copy(table_hbm.at[i1], r1, s1)

            return carry

        lax.fori_loop(0, n_pairs, body, 0)

    return sc_gather


def _make_tc_emit_chunk(Bt, L, Lh, l0, Bc, jb0, D_out, aliased):
    BB = 4096
    assert Bc % BB == 0
    nj = Bc // BB

    def select_t(x, h):
        xt = x.T
        lo = xt[:D_out, :]
        hi = xt[D_out:, :]
        return (lo + h * (hi - lo))[None]

    if aliased:
        def body(prev_ref, x_ref, h_ref, o_ref):
            o_ref[...] = select_t(x_ref[...], h_ref[0])

        in_specs = [
            pl.BlockSpec((1, 8, 128), lambda l, j: (0, 0, 0)),
            pl.BlockSpec((BB, 2 * D_out), lambda l, j: (l * nj + j, 0)),
            pl.BlockSpec((1, 1, BB), lambda l, j: (l + l0, 0, jb0 + j)),
        ]
        io_alias = {0: 0}
    else:
        def body(x_ref, h_ref, o_ref):
            o_ref[...] = select_t(x_ref[...], h_ref[0])

        in_specs = [
            pl.BlockSpec((BB, 2 * D_out), lambda l, j: (l * nj + j, 0)),
            pl.BlockSpec((1, 1, BB), lambda l, j: (l + l0, 0, jb0 + j)),
        ]
        io_alias = {}

    return pl.pallas_call(
        body,
        grid=(Lh, nj),
        in_specs=in_specs,
        out_specs=pl.BlockSpec((1, D_out, BB),
                               lambda l, j: (l + l0, 0, jb0 + j)),
        out_shape=jax.ShapeDtypeStruct((L, D_out, Bt), jnp.float32),
        input_output_aliases=io_alias,
        compiler_params=pltpu.CompilerParams(
            dimension_semantics=("arbitrary", "arbitrary"),
        ),
    )


def kernel(indices, table, W, b):
    Bt, L = indices.shape
    V, D = table.shape
    D_out = W.shape[1]
    Lh = L // 2
    Bc = Bt // 2
    Bh = Bc * Lh
    split = (V // 2 // _BK) * _BK
    tT = table.T
    zero = jnp.zeros_like(W)
    W4 = jnp.concatenate(
        [jnp.concatenate([W, zero], axis=1),
         jnp.concatenate([zero, W], axis=1)], axis=0)
    b2 = jnp.concatenate([b, b]).reshape(1, 2 * D_out)
    twb = _make_tc_transform(V, D, D_out, split)(tT, tT, W4, b2)
    idxT = indices.T.astype(jnp.int32)
    idx2 = jnp.where(idxT < split, idxT, idxT - split)
    half = (idxT >= split).astype(jnp.float32).reshape(L, 1, Bt)
    gather = _make_sc_gather(2 * D_out, Bh)
    chunks = [(l0, b0) for l0 in (0, Lh) for b0 in (0, Bc)]
    gs = [gather(twb,
                 lax.slice(idx2, (l0, b0), (l0 + Lh, b0 + Bc)).reshape(Bh))
          for l0, b0 in chunks]
    out = None
    for k, (l0, b0) in enumerate(chunks):
        emit = _make_tc_emit_chunk(Bt, L, Lh, l0, Bc, b0 // 4096, D_out,
                                   aliased=(k > 0))
        out = emit(gs[k], half) if k == 0 else emit(out, gs[k], half)
    return jnp.transpose(out, (2, 0, 1))
